# baseline (speedup 1.0000x reference)
import functools

import jax
import jax.numpy as jnp
from jax import lax
from jax.experimental import pallas as pl
from jax.experimental.pallas import tpu as pltpu
from jax.experimental.pallas import tpu_sc as plsc

EMB = 128
VOCAB = 84
NUM_ROWS = 16384 * 200
NUM_WORKERS = 32
ROWS_PER_WORKER = NUM_ROWS // NUM_WORKERS
CHUNK = 400
G = 240
E = CHUNK - G
EG = E // 16
STEPS = ROWS_PER_WORKER // CHUNK
J = STEPS // 2


def _sc_body(idx_hbm, table_hbm, out_hbm, table_sp, table_tile,
             idx0, idx1, rg0, rg1, re0, re1,
             isem0, isem1, gsem0, gsem1, sgsem0, sgsem1, sesem0, sesem1):
    idx = (idx0, idx1)
    rows_g = (rg0, rg1)
    rows_e = (re0, re1)
    isem = (isem0, isem1)
    gsem = (gsem0, gsem1)
    sgsem = (sgsem0, sgsem1)
    sesem = (sesem0, sesem1)

    sid = lax.axis_index("s")
    wid = sid * 2 + lax.axis_index("c")
    base = wid * ROWS_PER_WORKER

    pltpu.sync_copy(table_hbm, table_tile)

    @pl.when(sid == 0)
    def _stage():
        pltpu.sync_copy(table_hbm, rg0.at[pl.ds(0, VOCAB)])
        pltpu.sync_copy(rg0.at[pl.ds(0, VOCAB)], table_sp)

    plsc.subcore_barrier()

    iota = lax.iota(jnp.int32, 16)
    colj = [(iota + j) & 15 for j in range(16)]

    def idx_start(i, s):
        pltpu.async_copy(idx_hbm.at[pl.ds(base + i * CHUNK, CHUNK)],
                         idx[s], isem[s])

    def idx_wait(s):
        pltpu.make_async_copy(idx_hbm.at[pl.ds(0, CHUNK)], idx[s], isem[s]).wait()

    def gather_start(s):
        pltpu.async_copy(table_sp.at[idx[s].at[pl.ds(0, G)]], rows_g[s], gsem[s])

    def gather_wait(s):
        pltpu.make_async_copy(table_sp.at[idx[s].at[pl.ds(0, G)]],
                              rows_g[s], gsem[s]).wait()

    def store_g_start(i, s):
        pltpu.async_copy(rows_g[s], out_hbm.at[pl.ds(base + i * CHUNK, G)],
                         sgsem[s])

    def store_g_wait(s):
        pltpu.make_async_copy(rows_g[s], out_hbm.at[pl.ds(0, G)], sgsem[s]).wait()

    def store_e_start(i, s):
        pltpu.async_copy(rows_e[s], out_hbm.at[pl.ds(base + i * CHUNK + G, E)],
                         sesem[s])

    def store_e_wait(s):
        pltpu.make_async_copy(rows_e[s], out_hbm.at[pl.ds(0, E)], sesem[s]).wait()

    def expand(s):
        def group(g, carry):
            rowv = idx[s][pl.ds(G + g * 16, 16)]
            outv = jnp.full((16,), g * 16, dtype=jnp.int32) + iota
            for c in range(8):
                for j in range(16):
                    cv = colj[j] + (c * 16)
                    val = plsc.load_gather(table_tile, [rowv, cv])
                    plsc.store_scatter(rows_e[s], [outv, cv], val)
            return carry

        lax.fori_loop(0, EG, group, 0)

    idx_start(0, 0)
    idx_start(1, 1)
    idx_wait(0)
    gather_start(0)

    def block(jb, carry):
        for s in range(2):
            i = 2 * jb + s
            s1 = 1 - s
            gather_wait(s)
            store_g_start(i, s)

            if s == 0:
                @pl.when(jb >= 1)
                def _():
                    store_g_wait(s1)

                idx_wait(s1)
                gather_start(s1)
            else:
                @pl.when(jb < J - 1)
                def _():
                    store_g_wait(s1)
                    idx_wait(s1)
                    gather_start(s1)

            @pl.when(jb >= 1)
            def _():
                store_e_wait(s)

            expand(s)
            store_e_start(i, s)

            @pl.when(jb < J - 1)
            def _():
                idx_start(i + 2, s)

        return carry

    lax.fori_loop(0, J, block, 0)

    for s in range(2):
        store_g_wait(s)
        store_e_wait(s)


_sc_gather = functools.partial(
    pl.kernel,
    mesh=plsc.VectorSubcoreMesh(core_axis_name="c", subcore_axis_name="s"),
    out_type=jax.ShapeDtypeStruct((NUM_ROWS, EMB), jnp.float32),
    compiler_params=pltpu.CompilerParams(needs_layout_passes=False),
    scratch_types=(
        [pltpu.VMEM_SHARED((VOCAB, EMB), jnp.float32),
         pltpu.VMEM((VOCAB, EMB), jnp.float32)]
        + [pltpu.VMEM((CHUNK,), jnp.int32) for _ in range(2)]
        + [pltpu.VMEM((G, EMB), jnp.float32) for _ in range(2)]
        + [pltpu.VMEM((E, EMB), jnp.float32) for _ in range(2)]
        + [pltpu.SemaphoreType.DMA for _ in range(8)]
    ),
)(_sc_body)


def kernel(x, table):
    flat = _sc_gather(x.reshape(-1), table)
    return flat.reshape(x.shape + (EMB,))


# speedup vs baseline: 1.6792x; 1.6792x over previous
import functools

import jax
import jax.numpy as jnp
from jax import lax
from jax.experimental import pallas as pl
from jax.experimental.pallas import tpu as pltpu
from jax.experimental.pallas import tpu_sc as plsc

EMB = 128
VOCAB = 84
NUM_ROWS = 16384 * 200
NUM_WORKERS = 32
ROWS_PER_WORKER = NUM_ROWS // NUM_WORKERS
CHUNK = 400
G = 320
E = CHUNK - G
EG = E // 16
STEPS = ROWS_PER_WORKER // CHUNK
J = STEPS // 2


def _sc_body(idx_hbm, table_hbm, out_hbm, table_sp, table_tile,
             idx0, idx1, rg0, rg1, re0, re1,
             isem0, isem1, gsem0, gsem1, sgsem0, sgsem1, sesem0, sesem1):
    idx = (idx0, idx1)
    rows_g = (rg0, rg1)
    rows_e = (re0, re1)
    isem = (isem0, isem1)
    gsem = (gsem0, gsem1)
    sgsem = (sgsem0, sgsem1)
    sesem = (sesem0, sesem1)

    sid = lax.axis_index("s")
    wid = sid * 2 + lax.axis_index("c")
    base = wid * ROWS_PER_WORKER

    pltpu.sync_copy(table_hbm, table_tile)

    @pl.when(sid == 0)
    def _stage():
        pltpu.sync_copy(table_hbm, rg0.at[pl.ds(0, VOCAB)])
        pltpu.sync_copy(rg0.at[pl.ds(0, VOCAB)], table_sp)

    plsc.subcore_barrier()

    iota = lax.iota(jnp.int32, 16)
    colj = [(iota + j) & 15 for j in range(16)]

    def idx_start(i, s):
        pltpu.async_copy(idx_hbm.at[pl.ds(base + i * CHUNK, CHUNK)],
                         idx[s], isem[s])

    def idx_wait(s):
        pltpu.make_async_copy(idx_hbm.at[pl.ds(0, CHUNK)], idx[s], isem[s]).wait()

    def gather_start(s):
        pltpu.async_copy(table_sp.at[idx[s].at[pl.ds(0, G)]], rows_g[s], gsem[s])

    def gather_wait(s):
        pltpu.make_async_copy(table_sp.at[idx[s].at[pl.ds(0, G)]],
                              rows_g[s], gsem[s]).wait()

    def store_g_start(i, s):
        pltpu.async_copy(rows_g[s], out_hbm.at[pl.ds(base + i * CHUNK, G)],
                         sgsem[s])

    def store_g_wait(s):
        pltpu.make_async_copy(rows_g[s], out_hbm.at[pl.ds(0, G)], sgsem[s]).wait()

    def store_e_start(i, s):
        pltpu.async_copy(rows_e[s], out_hbm.at[pl.ds(base + i * CHUNK + G, E)],
                         sesem[s])

    def store_e_wait(s):
        pltpu.make_async_copy(rows_e[s], out_hbm.at[pl.ds(0, E)], sesem[s]).wait()

    def expand(s):
        def group(g, carry):
            rowv = idx[s][pl.ds(G + g * 16, 16)]
            outv = jnp.full((16,), g * 16, dtype=jnp.int32) + iota
            for c in range(8):
                for j in range(16):
                    cv = colj[j] + (c * 16)
                    val = plsc.load_gather(table_tile, [rowv, cv])
                    plsc.store_scatter(rows_e[s], [outv, cv], val)
            return carry

        lax.fori_loop(0, EG, group, 0)

    idx_start(0, 0)
    idx_start(1, 1)
    idx_wait(0)
    gather_start(0)

    def block(jb, carry):
        for s in range(2):
            i = 2 * jb + s
            s1 = 1 - s
            gather_wait(s)
            store_g_start(i, s)

            if s == 0:
                @pl.when(jb >= 1)
                def _():
                    store_g_wait(s1)

                idx_wait(s1)
                gather_start(s1)
            else:
                @pl.when(jb < J - 1)
                def _():
                    store_g_wait(s1)
                    idx_wait(s1)
                    gather_start(s1)

            @pl.when(jb >= 1)
            def _():
                store_e_wait(s)

            expand(s)
            store_e_start(i, s)

            @pl.when(jb < J - 1)
            def _():
                idx_start(i + 2, s)

        return carry

    lax.fori_loop(0, J, block, 0)

    for s in range(2):
        store_g_wait(s)
        store_e_wait(s)


_sc_gather = functools.partial(
    pl.kernel,
    mesh=plsc.VectorSubcoreMesh(core_axis_name="c", subcore_axis_name="s"),
    out_type=jax.ShapeDtypeStruct((NUM_ROWS, EMB), jnp.float32),
    compiler_params=pltpu.CompilerParams(needs_layout_passes=False),
    scratch_types=(
        [pltpu.VMEM_SHARED((VOCAB, EMB), jnp.float32),
         pltpu.VMEM((VOCAB, EMB), jnp.float32)]
        + [pltpu.VMEM((CHUNK,), jnp.int32) for _ in range(2)]
        + [pltpu.VMEM((G, EMB), jnp.float32) for _ in range(2)]
        + [pltpu.VMEM((E, EMB), jnp.float32) for _ in range(2)]
        + [pltpu.SemaphoreType.DMA for _ in range(8)]
    ),
)(_sc_body)


def kernel(x, table):
    flat = _sc_gather(x.reshape(-1), table)
    return flat.reshape(x.shape + (EMB,))


# speedup vs baseline: 2.1177x; 1.2611x over previous
import functools

import jax
import jax.numpy as jnp
from jax import lax
from jax.experimental import pallas as pl
from jax.experimental.pallas import tpu as pltpu
from jax.experimental.pallas import tpu_sc as plsc

EMB = 128
VOCAB = 84
NUM_ROWS = 16384 * 200
NUM_WORKERS = 32
ROWS_PER_WORKER = NUM_ROWS // NUM_WORKERS
NBUF = 2
CHUNK = 400
STEPS = ROWS_PER_WORKER // CHUNK
JB = STEPS // NBUF


def _sc_body(idx_hbm, table_hbm, out_hbm, table_sp, *bufs):
    idx = bufs[0:NBUF]
    rows = bufs[NBUF:2 * NBUF]
    isem = bufs[2 * NBUF:3 * NBUF]
    gsem = bufs[3 * NBUF:4 * NBUF]
    ssem = bufs[4 * NBUF:5 * NBUF]

    sid = lax.axis_index("s")
    wid = sid * 2 + lax.axis_index("c")
    base = wid * ROWS_PER_WORKER

    @pl.when(sid == 0)
    def _stage():
        pltpu.sync_copy(table_hbm, rows[0].at[pl.ds(0, VOCAB)])
        pltpu.sync_copy(rows[0].at[pl.ds(0, VOCAB)], table_sp)

    plsc.subcore_barrier()

    def idx_start(i, s):
        pltpu.async_copy(idx_hbm.at[pl.ds(base + i * CHUNK, CHUNK)],
                         idx[s], isem[s])

    def idx_wait(s):
        pltpu.make_async_copy(idx_hbm.at[pl.ds(0, CHUNK)], idx[s], isem[s]).wait()

    def gather_start(s):
        pltpu.async_copy(table_sp.at[idx[s]], rows[s], gsem[s])

    def gather_wait(s):
        pltpu.make_async_copy(table_sp.at[idx[s]], rows[s], gsem[s]).wait()

    def store_start(i, s):
        pltpu.async_copy(rows[s], out_hbm.at[pl.ds(base + i * CHUNK, CHUNK)],
                         ssem[s])

    def store_wait(s):
        pltpu.make_async_copy(rows[s], out_hbm.at[pl.ds(0, CHUNK)], ssem[s]).wait()

    for s in range(NBUF):
        idx_start(s, s)
    idx_wait(0)
    gather_start(0)

    def block(jb, carry):
        i0 = NBUF * jb
        for s in range(NBUF):
            gather_wait(s)
            store_start(i0 + s, s)

            @pl.when(jb < JB - 1)
            def _():
                idx_start(i0 + s + NBUF, s)

            s1 = (s + 1) % NBUF
            if s < NBUF - 1:
                idx_wait(s1)

                @pl.when(jb >= 1)
                def _():
                    store_wait(s1)

                gather_start(s1)
            else:
                @pl.when(jb < JB - 1)
                def _():
                    idx_wait(s1)
                    store_wait(s1)
                    gather_start(s1)

        return carry

    lax.fori_loop(0, JB, block, 0)

    for s in range(NBUF):
        store_wait(s)


_sc_gather = functools.partial(
    pl.kernel,
    mesh=plsc.VectorSubcoreMesh(core_axis_name="c", subcore_axis_name="s"),
    out_type=jax.ShapeDtypeStruct((NUM_ROWS, EMB), jnp.float32),
    compiler_params=pltpu.CompilerParams(
        disable_bounds_checks=True, disable_semaphore_checks=True),
    scratch_types=(
        [pltpu.VMEM_SHARED((VOCAB, EMB), jnp.float32)]
        + [pltpu.VMEM((CHUNK,), jnp.int32) for _ in range(NBUF)]
        + [pltpu.VMEM((CHUNK, EMB), jnp.float32) for _ in range(NBUF)]
        + [pltpu.SemaphoreType.DMA for _ in range(3 * NBUF)]
    ),
)(_sc_body)


def kernel(x, table):
    flat = _sc_gather(x.reshape(-1), table)
    return flat.reshape(x.shape + (EMB,))
